# Initial kernel scaffold; baseline (speedup 1.0000x reference)
import jax
import jax.numpy as jnp
from jax.experimental import pallas as pl


def kernel(t_features, t_pseudo_labels, prototypes, delta_phi):
    raise NotImplementedError("write your pallas kernel here")


# speedup vs baseline: 4.0962x; 4.0962x over previous
import functools

import jax
import jax.numpy as jnp
from jax import lax
from jax.experimental import pallas as pl
from jax.experimental.pallas import tpu as pltpu
from jax.experimental.pallas import tpu_sc as plsc

_C = 10000
_D = 128
_N = 131072
_NC = 2
_NS = 16
_NW = _NC * _NS
_RPW = _N // _NW
_CHUNK = 128
_NCHUNK = _RPW // _CHUNK
_CPT = 624
_CREM = _C - _NS * _CPT


def _seg_body(feat_hbm, lab_hbm, zeros_hbm, psum_hbm, pcnt_hbm,
              idx_v, rows_v, accum_sh):
    cid = lax.axis_index("c")
    sid = lax.axis_index("s")
    wid = sid * _NC + cid

    ones16 = jnp.ones((16,), jnp.float32)

    def _zero_accum():
        pltpu.sync_copy(zeros_hbm, rows_v)
        for j in range(4):
            pltpu.sync_copy(
                rows_v, accum_sh.at[pl.ds(sid * _CPT + j * _CHUNK, _CHUNK)])
        pltpu.sync_copy(rows_v.at[pl.ds(0, 112)],
                        accum_sh.at[pl.ds(sid * _CPT + 512, 112)])

        @pl.when(sid == 0)
        def _():
            pltpu.sync_copy(rows_v.at[pl.ds(0, _CREM)],
                            accum_sh.at[pl.ds(_NS * _CPT, _CREM)])

    def _copy_out(dst_hbm):
        def _out(off, size):
            pltpu.sync_copy(accum_sh.at[pl.ds(off, size)],
                            rows_v.at[pl.ds(0, size)])
            pltpu.sync_copy(rows_v.at[pl.ds(0, size)],
                            dst_hbm.at[cid, pl.ds(off, size)])

        for j in range(4):
            _out(sid * _CPT + j * _CHUNK, _CHUNK)
        _out(sid * _CPT + 512, 112)

        @pl.when(sid == 0)
        def _():
            _out(_NS * _CPT, _CREM)

    _zero_accum()
    pltpu.sync_copy(lab_hbm.at[wid], idx_v)
    plsc.subcore_barrier()

    def _chunk(j, carry):
        pltpu.sync_copy(
            feat_hbm.at[pl.ds(wid * _RPW + j * _CHUNK, _CHUNK)], rows_v)
        pltpu.sync_copy(rows_v, accum_sh.at[idx_v.at[j]], add=True)
        return carry

    lax.fori_loop(0, _NCHUNK, _chunk, 0)
    plsc.subcore_barrier()
    _copy_out(psum_hbm)
    plsc.subcore_barrier()

    _zero_accum()
    plsc.subcore_barrier()

    def _fill_ones(i, carry):
        for t in range(_D // 16):
            rows_v[i, pl.ds(t * 16, 16)] = ones16
        return carry

    lax.fori_loop(0, _CHUNK, _fill_ones, 0)

    def _cchunk(j, carry):
        pltpu.sync_copy(rows_v, accum_sh.at[idx_v.at[j]], add=True)
        return carry

    lax.fori_loop(0, _NCHUNK, _cchunk, 0)
    plsc.subcore_barrier()
    _copy_out(pcnt_hbm)


@functools.cache
def _seg_call():
    return pl.kernel(
        _seg_body,
        out_type=(
            jax.ShapeDtypeStruct((_NC, _C, _D), jnp.float32),
            jax.ShapeDtypeStruct((_NC, _C, _D), jnp.float32),
        ),
        mesh=plsc.VectorSubcoreMesh(core_axis_name="c", subcore_axis_name="s",
                                    num_cores=_NC, num_subcores=_NS),
        scratch_types=[
            pltpu.VMEM((_NCHUNK, _CHUNK), jnp.int32),
            pltpu.VMEM((_CHUNK, _D), jnp.float32),
            pltpu.VMEM_SHARED((_C, _D), jnp.float32),
        ],
    )


_BLK = 1000


def _loss_body(psum_ref, pcnt_ref, proto_ref, dphi_ref, loss_ref, acc_ref):
    i = pl.program_id(0)

    @pl.when(i == 0)
    def _():
        acc_ref[0] = 0.0
        acc_ref[1] = 0.0

    sums = psum_ref[0] + psum_ref[1]
    counts = (pcnt_ref[0] + pcnt_ref[1])[:, 0]
    present = counts > 0.0
    means = sums / jnp.maximum(counts, 1.0)[:, None]
    diff = means - (proto_ref[...] + dphi_ref[...])
    norms = jnp.sqrt(jnp.sum(diff * diff, axis=1))
    acc_ref[0] += jnp.sum(jnp.where(present, norms, 0.0))
    acc_ref[1] += jnp.sum(present.astype(jnp.float32))

    @pl.when(i == pl.num_programs(0) - 1)
    def _():
        val = acc_ref[0] / jnp.maximum(acc_ref[1], 1.0)
        loss_ref[...] = jnp.broadcast_to(val, (1, 1))


def _loss_call(psum, pcnt, prototypes, delta_phi):
    return pl.pallas_call(
        _loss_body,
        grid=(_C // _BLK,),
        in_specs=[
            pl.BlockSpec((_NC, _BLK, _D), lambda i: (0, i, 0)),
            pl.BlockSpec((_NC, _BLK, _D), lambda i: (0, i, 0)),
            pl.BlockSpec((_BLK, _D), lambda i: (i, 0)),
            pl.BlockSpec((_BLK, _D), lambda i: (i, 0)),
        ],
        out_specs=pl.BlockSpec((1, 1), lambda i: (0, 0)),
        out_shape=jax.ShapeDtypeStruct((1, 1), jnp.float32),
        scratch_shapes=[pltpu.SMEM((2,), jnp.float32)],
    )(psum, pcnt, prototypes, delta_phi)


@jax.jit
def kernel(t_features, t_pseudo_labels, prototypes, delta_phi):
    labels = t_pseudo_labels.reshape(_NW, _NCHUNK, _CHUNK)
    zeros = jnp.zeros((_CHUNK, _D), jnp.float32)
    psum, pcnt = _seg_call()(t_features, labels, zeros)
    loss = _loss_call(psum, pcnt, prototypes, delta_phi)
    return loss[0, 0]


# speedup vs baseline: 4.2626x; 1.0406x over previous
import functools

import jax
import jax.numpy as jnp
from jax import lax
from jax.experimental import pallas as pl
from jax.experimental.pallas import tpu as pltpu
from jax.experimental.pallas import tpu_sc as plsc

_C = 10000
_D = 128
_N = 131072
_NC = 2
_NS = 16
_NW = _NC * _NS
_RPW = _N // _NW
_CHUNK = 128
_NCHUNK = _RPW // _CHUNK
_CPT = 624
_CREM = _C - _NS * _CPT


def _seg_body(feat_hbm, lab_hbm, zeros_hbm, psum_hbm, pcnt_hbm,
              idx_v, rows_v, accum_sh):
    cid = lax.axis_index("c")
    sid = lax.axis_index("s")
    wid = sid * _NC + cid

    ones16 = jnp.ones((16,), jnp.float32)

    def _zero_accum():
        pltpu.sync_copy(zeros_hbm, rows_v)
        for j in range(4):
            pltpu.sync_copy(
                rows_v, accum_sh.at[pl.ds(sid * _CPT + j * _CHUNK, _CHUNK)])
        pltpu.sync_copy(rows_v.at[pl.ds(0, 112)],
                        accum_sh.at[pl.ds(sid * _CPT + 512, 112)])

        @pl.when(sid == 0)
        def _():
            pltpu.sync_copy(rows_v.at[pl.ds(0, _CREM)],
                            accum_sh.at[pl.ds(_NS * _CPT, _CREM)])

    def _copy_out(dst_hbm):
        def _out(off, size):
            pltpu.sync_copy(accum_sh.at[pl.ds(off, size)],
                            rows_v.at[pl.ds(0, size)])
            pltpu.sync_copy(rows_v.at[pl.ds(0, size)],
                            dst_hbm.at[cid, pl.ds(off, size)])

        for j in range(4):
            _out(sid * _CPT + j * _CHUNK, _CHUNK)
        _out(sid * _CPT + 512, 112)

        @pl.when(sid == 0)
        def _():
            _out(_NS * _CPT, _CREM)

    _zero_accum()
    pltpu.sync_copy(lab_hbm.at[wid], idx_v)
    plsc.subcore_barrier()

    def _chunk(j, carry):
        pltpu.sync_copy(
            feat_hbm.at[pl.ds(wid * _RPW + j * _CHUNK, _CHUNK)], rows_v)
        pltpu.sync_copy(rows_v, accum_sh.at[idx_v.at[j]], add=True)
        return carry

    lax.fori_loop(0, _NCHUNK, _chunk, 0)
    plsc.subcore_barrier()
    _copy_out(psum_hbm)
    plsc.subcore_barrier()

    def _fill_ones(i, carry):
        for t in range(_D // 16):
            rows_v[i, pl.ds(t * 16, 16)] = ones16
        return carry

    lax.fori_loop(0, _CHUNK, _fill_ones, 0)

    def _cchunk(j, carry):
        pltpu.sync_copy(rows_v, accum_sh.at[idx_v.at[j]], add=True)
        return carry

    lax.fori_loop(0, _NCHUNK, _cchunk, 0)
    plsc.subcore_barrier()
    _copy_out(pcnt_hbm)


@functools.cache
def _seg_call():
    return pl.kernel(
        _seg_body,
        out_type=(
            jax.ShapeDtypeStruct((_NC, _C, _D), jnp.float32),
            jax.ShapeDtypeStruct((_NC, _C, _D), jnp.float32),
        ),
        mesh=plsc.VectorSubcoreMesh(core_axis_name="c", subcore_axis_name="s",
                                    num_cores=_NC, num_subcores=_NS),
        scratch_types=[
            pltpu.VMEM((_NCHUNK, _CHUNK), jnp.int32),
            pltpu.VMEM((_CHUNK, _D), jnp.float32),
            pltpu.VMEM_SHARED((_C, _D), jnp.float32),
        ],
    )


_BLK = 1000


def _loss_body(psum_ref, pcnt_ref, proto_ref, dphi_ref, loss_ref, acc_ref):
    i = pl.program_id(0)

    @pl.when(i == 0)
    def _():
        acc_ref[0] = 0.0
        acc_ref[1] = 0.0

    sums = psum_ref[0] + psum_ref[1]
    counts = (pcnt_ref[0] - psum_ref[0]
              + pcnt_ref[1] - psum_ref[1])[:, 0]
    present = counts > 0.0
    means = sums / jnp.maximum(counts, 1.0)[:, None]
    diff = means - (proto_ref[...] + dphi_ref[...])
    norms = jnp.sqrt(jnp.sum(diff * diff, axis=1))
    acc_ref[0] += jnp.sum(jnp.where(present, norms, 0.0))
    acc_ref[1] += jnp.sum(present.astype(jnp.float32))

    @pl.when(i == pl.num_programs(0) - 1)
    def _():
        val = acc_ref[0] / jnp.maximum(acc_ref[1], 1.0)
        loss_ref[...] = jnp.broadcast_to(val, (1, 1))


def _loss_call(psum, pcnt, prototypes, delta_phi):
    return pl.pallas_call(
        _loss_body,
        grid=(_C // _BLK,),
        in_specs=[
            pl.BlockSpec((_NC, _BLK, _D), lambda i: (0, i, 0)),
            pl.BlockSpec((_NC, _BLK, _D), lambda i: (0, i, 0)),
            pl.BlockSpec((_BLK, _D), lambda i: (i, 0)),
            pl.BlockSpec((_BLK, _D), lambda i: (i, 0)),
        ],
        out_specs=pl.BlockSpec((1, 1), lambda i: (0, 0)),
        out_shape=jax.ShapeDtypeStruct((1, 1), jnp.float32),
        scratch_shapes=[pltpu.SMEM((2,), jnp.float32)],
    )(psum, pcnt, prototypes, delta_phi)


@jax.jit
def kernel(t_features, t_pseudo_labels, prototypes, delta_phi):
    labels = t_pseudo_labels.reshape(_NW, _NCHUNK, _CHUNK)
    zeros = jnp.zeros((_CHUNK, _D), jnp.float32)
    psum, pcnt = _seg_call()(t_features, labels, zeros)
    loss = _loss_call(psum, pcnt, prototypes, delta_phi)
    return loss[0, 0]


# speedup vs baseline: 4.7912x; 1.1240x over previous
import functools

import jax
import jax.numpy as jnp
from jax import lax
from jax.experimental import pallas as pl
from jax.experimental.pallas import tpu as pltpu
from jax.experimental.pallas import tpu_sc as plsc

_C = 10000
_D = 128
_N = 131072
_NC = 2
_NS = 16
_NW = _NC * _NS
_RPW = _N // _NW
_CHUNK = 128
_NCHUNK = _RPW // _CHUNK
_CPT = 624
_CREM = _C - _NS * _CPT


_HC = 64
_NHC = _RPW // _HC


def _seg_body(feat_hbm, lab_hbm, zeros_hbm, psum_hbm, pcnt_hbm,
              idx_v, rows_a, rows_b, accum_sh, sem_a, sem_b):
    cid = lax.axis_index("c")
    sid = lax.axis_index("s")
    wid = sid * _NC + cid

    zeros16 = jnp.zeros((16,), jnp.float32)
    ones16 = jnp.ones((16,), jnp.float32)

    def _feat(h):
        return feat_hbm.at[pl.ds(wid * _RPW + h * _HC, _HC)]

    pltpu.sync_copy(zeros_hbm, rows_a)
    for j in range(9):
        pltpu.sync_copy(
            rows_a, accum_sh.at[pl.ds(sid * _CPT + j * _HC, _HC)])
    pltpu.sync_copy(rows_a.at[pl.ds(0, 48)],
                    accum_sh.at[pl.ds(sid * _CPT + 576, 48)])

    @pl.when(sid == 0)
    def _():
        pltpu.sync_copy(rows_a.at[pl.ds(0, _CREM)],
                        accum_sh.at[pl.ds(_NS * _CPT, _CREM)])

    pltpu.sync_copy(lab_hbm.at[wid], idx_v)
    plsc.subcore_barrier()

    pltpu.async_copy(_feat(0), rows_a, sem_a)

    def _pair(i, carry):
        h0 = 2 * i
        pltpu.async_copy(_feat(h0 + 1), rows_b, sem_b)
        pltpu.make_async_copy(_feat(h0), rows_a, sem_a).wait()
        pltpu.sync_copy(rows_a, accum_sh.at[idx_v.at[h0]], add=True)

        @pl.when(i + 1 < _NHC // 2)
        def _():
            pltpu.async_copy(_feat(h0 + 2), rows_a, sem_a)

        pltpu.make_async_copy(_feat(h0 + 1), rows_b, sem_b).wait()
        pltpu.sync_copy(rows_b, accum_sh.at[idx_v.at[h0 + 1]], add=True)
        return carry

    lax.fori_loop(0, _NHC // 2, _pair, 0)
    plsc.subcore_barrier()

    def _copy_out(dst_hbm):
        def _out(off, size):
            pltpu.sync_copy(accum_sh.at[pl.ds(off, size)],
                            rows_a.at[pl.ds(0, size)])
            pltpu.sync_copy(rows_a.at[pl.ds(0, size)],
                            dst_hbm.at[cid, pl.ds(off, size)])

        for j in range(9):
            _out(sid * _CPT + j * _HC, _HC)
        _out(sid * _CPT + 576, 48)

        @pl.when(sid == 0)
        def _():
            _out(_NS * _CPT, _CREM)

    _copy_out(psum_hbm)
    plsc.subcore_barrier()

    def _fill_ones(i, carry):
        for t in range(_D // 16):
            rows_a[i, pl.ds(t * 16, 16)] = ones16
        return carry

    lax.fori_loop(0, _HC, _fill_ones, 0)

    def _cchunk(h, carry):
        pltpu.sync_copy(rows_a, accum_sh.at[idx_v.at[h]], add=True)
        return carry

    lax.fori_loop(0, _NHC, _cchunk, 0)
    plsc.subcore_barrier()
    _copy_out(pcnt_hbm)


@functools.cache
def _seg_call():
    return pl.kernel(
        _seg_body,
        out_type=(
            jax.ShapeDtypeStruct((_NC, _C, _D), jnp.float32),
            jax.ShapeDtypeStruct((_NC, _C, _D), jnp.float32),
        ),
        mesh=plsc.VectorSubcoreMesh(core_axis_name="c", subcore_axis_name="s",
                                    num_cores=_NC, num_subcores=_NS),
        scratch_types=[
            pltpu.VMEM((_NHC, _HC), jnp.int32),
            pltpu.VMEM((_HC, _D), jnp.float32),
            pltpu.VMEM((_HC, _D), jnp.float32),
            pltpu.VMEM_SHARED((_C, _D), jnp.float32),
            pltpu.SemaphoreType.DMA,
            pltpu.SemaphoreType.DMA,
        ],
    )


_BLK = 1000


def _loss_body(psum_ref, pcnt_ref, proto_ref, dphi_ref, loss_ref, acc_ref):
    i = pl.program_id(0)

    @pl.when(i == 0)
    def _():
        acc_ref[0] = 0.0
        acc_ref[1] = 0.0

    sums = psum_ref[0] + psum_ref[1]
    counts = (pcnt_ref[0] - psum_ref[0]
              + pcnt_ref[1] - psum_ref[1])[:, 0]
    present = counts > 0.0
    means = sums / jnp.maximum(counts, 1.0)[:, None]
    diff = means - (proto_ref[...] + dphi_ref[...])
    norms = jnp.sqrt(jnp.sum(diff * diff, axis=1))
    acc_ref[0] += jnp.sum(jnp.where(present, norms, 0.0))
    acc_ref[1] += jnp.sum(present.astype(jnp.float32))

    @pl.when(i == pl.num_programs(0) - 1)
    def _():
        val = acc_ref[0] / jnp.maximum(acc_ref[1], 1.0)
        loss_ref[...] = jnp.broadcast_to(val, (1, 1))


def _loss_call(psum, pcnt, prototypes, delta_phi):
    return pl.pallas_call(
        _loss_body,
        grid=(_C // _BLK,),
        in_specs=[
            pl.BlockSpec((_NC, _BLK, _D), lambda i: (0, i, 0)),
            pl.BlockSpec((_NC, _BLK, _D), lambda i: (0, i, 0)),
            pl.BlockSpec((_BLK, _D), lambda i: (i, 0)),
            pl.BlockSpec((_BLK, _D), lambda i: (i, 0)),
        ],
        out_specs=pl.BlockSpec((1, 1), lambda i: (0, 0)),
        out_shape=jax.ShapeDtypeStruct((1, 1), jnp.float32),
        scratch_shapes=[pltpu.SMEM((2,), jnp.float32)],
    )(psum, pcnt, prototypes, delta_phi)


@jax.jit
def kernel(t_features, t_pseudo_labels, prototypes, delta_phi):
    labels = t_pseudo_labels.reshape(_NW, _NHC, _HC)
    zeros = jnp.zeros((_HC, _D), jnp.float32)
    psum, pcnt = _seg_call()(t_features, labels, zeros)
    loss = _loss_call(psum, pcnt, prototypes, delta_phi)
    return loss[0, 0]


# speedup vs baseline: 4.8328x; 1.0087x over previous
import functools

import jax
import jax.numpy as jnp
from jax import lax
from jax.experimental import pallas as pl
from jax.experimental.pallas import tpu as pltpu
from jax.experimental.pallas import tpu_sc as plsc

_C = 10000
_D = 128
_N = 131072
_NC = 2
_NS = 16
_NW = _NC * _NS
_RPW = _N // _NW
_CHUNK = 128
_NCHUNK = _RPW // _CHUNK
_CPT = 624
_CREM = _C - _NS * _CPT


_HC = 64
_NHC = _RPW // _HC


def _seg_body(feat_hbm, lab_hbm, zeros_hbm, psum_hbm, pcnt_hbm,
              idx_v, rows_a, rows_b, accum_sh, sem_a, sem_b):
    cid = lax.axis_index("c")
    sid = lax.axis_index("s")
    wid = sid * _NC + cid

    zeros16 = jnp.zeros((16,), jnp.float32)
    ones16 = jnp.ones((16,), jnp.float32)

    def _feat(h):
        return feat_hbm.at[pl.ds(wid * _RPW + h * _HC, _HC)]

    pltpu.sync_copy(zeros_hbm, rows_a)
    for j in range(9):
        pltpu.sync_copy(
            rows_a, accum_sh.at[pl.ds(sid * _CPT + j * _HC, _HC)])
    pltpu.sync_copy(rows_a.at[pl.ds(0, 48)],
                    accum_sh.at[pl.ds(sid * _CPT + 576, 48)])

    @pl.when(sid == 0)
    def _():
        pltpu.sync_copy(rows_a.at[pl.ds(0, _CREM)],
                        accum_sh.at[pl.ds(_NS * _CPT, _CREM)])

    pltpu.sync_copy(lab_hbm.at[wid], idx_v)
    plsc.subcore_barrier()

    pltpu.async_copy(_feat(0), rows_a, sem_a)

    def _pair(i, carry):
        h0 = 2 * i
        pltpu.async_copy(_feat(h0 + 1), rows_b, sem_b)
        pltpu.make_async_copy(_feat(h0), rows_a, sem_a).wait()
        pltpu.sync_copy(rows_a, accum_sh.at[idx_v.at[h0]], add=True)

        @pl.when(i + 1 < _NHC // 2)
        def _():
            pltpu.async_copy(_feat(h0 + 2), rows_a, sem_a)

        pltpu.make_async_copy(_feat(h0 + 1), rows_b, sem_b).wait()
        pltpu.sync_copy(rows_b, accum_sh.at[idx_v.at[h0 + 1]], add=True)
        return carry

    lax.fori_loop(0, _NHC // 2, _pair, 0)
    plsc.subcore_barrier()

    def _copy_out(dst_hbm):
        def _out(off, size):
            pltpu.sync_copy(accum_sh.at[pl.ds(off, size)],
                            rows_a.at[pl.ds(0, size)])
            pltpu.sync_copy(rows_a.at[pl.ds(0, size)],
                            dst_hbm.at[cid, pl.ds(off, size)])

        for j in range(9):
            _out(sid * _CPT + j * _HC, _HC)
        _out(sid * _CPT + 576, 48)

        @pl.when(sid == 0)
        def _():
            _out(_NS * _CPT, _CREM)

    _copy_out(psum_hbm)
    plsc.subcore_barrier()

    def _fill_ones(i, carry):
        for t in range(_D // 16):
            rows_a[i, pl.ds(t * 16, 16)] = ones16
        return carry

    lax.fori_loop(0, _HC, _fill_ones, 0)

    def _cfire(h, carry):
        pltpu.async_copy(rows_a, accum_sh.at[idx_v.at[h]], sem_a, add=True)
        return carry

    lax.fori_loop(0, _NHC, _cfire, 0)

    def _cdrain(h, carry):
        pltpu.make_async_copy(rows_a, accum_sh.at[idx_v.at[0]], sem_a).wait()
        return carry

    lax.fori_loop(0, _NHC, _cdrain, 0)
    plsc.subcore_barrier()
    _copy_out(pcnt_hbm)


@functools.cache
def _seg_call():
    return pl.kernel(
        _seg_body,
        out_type=(
            jax.ShapeDtypeStruct((_NC, _C, _D), jnp.float32),
            jax.ShapeDtypeStruct((_NC, _C, _D), jnp.float32),
        ),
        mesh=plsc.VectorSubcoreMesh(core_axis_name="c", subcore_axis_name="s",
                                    num_cores=_NC, num_subcores=_NS),
        scratch_types=[
            pltpu.VMEM((_NHC, _HC), jnp.int32),
            pltpu.VMEM((_HC, _D), jnp.float32),
            pltpu.VMEM((_HC, _D), jnp.float32),
            pltpu.VMEM_SHARED((_C, _D), jnp.float32),
            pltpu.SemaphoreType.DMA,
            pltpu.SemaphoreType.DMA,
        ],
    )


_BLK = 1000


def _loss_body(psum_ref, pcnt_ref, proto_ref, dphi_ref, loss_ref, acc_ref):
    i = pl.program_id(0)

    @pl.when(i == 0)
    def _():
        acc_ref[0] = 0.0
        acc_ref[1] = 0.0

    sums = psum_ref[0] + psum_ref[1]
    counts = (pcnt_ref[0] - psum_ref[0]
              + pcnt_ref[1] - psum_ref[1])[:, 0]
    present = counts > 0.0
    means = sums / jnp.maximum(counts, 1.0)[:, None]
    diff = means - (proto_ref[...] + dphi_ref[...])
    norms = jnp.sqrt(jnp.sum(diff * diff, axis=1))
    acc_ref[0] += jnp.sum(jnp.where(present, norms, 0.0))
    acc_ref[1] += jnp.sum(present.astype(jnp.float32))

    @pl.when(i == pl.num_programs(0) - 1)
    def _():
        val = acc_ref[0] / jnp.maximum(acc_ref[1], 1.0)
        loss_ref[...] = jnp.broadcast_to(val, (1, 1))


def _loss_call(psum, pcnt, prototypes, delta_phi):
    return pl.pallas_call(
        _loss_body,
        grid=(_C // _BLK,),
        in_specs=[
            pl.BlockSpec((_NC, _BLK, _D), lambda i: (0, i, 0)),
            pl.BlockSpec((_NC, _BLK, _D), lambda i: (0, i, 0)),
            pl.BlockSpec((_BLK, _D), lambda i: (i, 0)),
            pl.BlockSpec((_BLK, _D), lambda i: (i, 0)),
        ],
        out_specs=pl.BlockSpec((1, 1), lambda i: (0, 0)),
        out_shape=jax.ShapeDtypeStruct((1, 1), jnp.float32),
        scratch_shapes=[pltpu.SMEM((2,), jnp.float32)],
    )(psum, pcnt, prototypes, delta_phi)


@jax.jit
def kernel(t_features, t_pseudo_labels, prototypes, delta_phi):
    labels = t_pseudo_labels.reshape(_NW, _NHC, _HC)
    zeros = jnp.zeros((_HC, _D), jnp.float32)
    psum, pcnt = _seg_call()(t_features, labels, zeros)
    loss = _loss_call(psum, pcnt, prototypes, delta_phi)
    return loss[0, 0]


# speedup vs baseline: 5.1286x; 1.0612x over previous
import functools

import jax
import jax.numpy as jnp
from jax import lax
from jax.experimental import pallas as pl
from jax.experimental.pallas import tpu as pltpu
from jax.experimental.pallas import tpu_sc as plsc

_C = 10000
_D = 128
_N = 131072
_NC = 2
_NS = 16
_NW = _NC * _NS
_RPW = _N // _NW
_CHUNK = 128
_NCHUNK = _RPW // _CHUNK
_CPT = 624
_CREM = _C - _NS * _CPT


_HC = 64
_NHC = _RPW // _HC


def _seg_body(feat_hbm, lab_hbm, zeros_hbm, psum_hbm, pcnt_hbm,
              idx_v, rows_a, rows_b, accum_sh, sem_a, sem_b):
    cid = lax.axis_index("c")
    sid = lax.axis_index("s")
    wid = sid * _NC + cid

    zeros16 = jnp.zeros((16,), jnp.float32)
    ones16 = jnp.ones((16,), jnp.float32)

    def _feat(h):
        return feat_hbm.at[pl.ds(wid * _RPW + h * _HC, _HC)]

    pltpu.sync_copy(zeros_hbm, rows_a)
    for j in range(9):
        pltpu.async_copy(
            rows_a, accum_sh.at[pl.ds(sid * _CPT + j * _HC, _HC)], sem_b)
    pltpu.async_copy(rows_a.at[pl.ds(0, 48)],
                     accum_sh.at[pl.ds(sid * _CPT + 576, 48)], sem_b)

    @pl.when(sid == 0)
    def _():
        pltpu.async_copy(rows_a.at[pl.ds(0, _CREM)],
                         accum_sh.at[pl.ds(_NS * _CPT, _CREM)], sem_b)

    pltpu.sync_copy(lab_hbm.at[wid], idx_v)

    for j in range(9):
        pltpu.make_async_copy(
            rows_a, accum_sh.at[pl.ds(sid * _CPT + j * _HC, _HC)],
            sem_b).wait()
    pltpu.make_async_copy(rows_a.at[pl.ds(0, 48)],
                          accum_sh.at[pl.ds(sid * _CPT + 576, 48)],
                          sem_b).wait()

    @pl.when(sid == 0)
    def _():
        pltpu.make_async_copy(rows_a.at[pl.ds(0, _CREM)],
                              accum_sh.at[pl.ds(_NS * _CPT, _CREM)],
                              sem_b).wait()

    plsc.subcore_barrier()

    pltpu.async_copy(_feat(0), rows_a, sem_a)

    def _pair(i, carry):
        h0 = 2 * i
        pltpu.async_copy(_feat(h0 + 1), rows_b, sem_b)
        pltpu.make_async_copy(_feat(h0), rows_a, sem_a).wait()
        pltpu.sync_copy(rows_a, accum_sh.at[idx_v.at[h0]], add=True)

        @pl.when(i + 1 < _NHC // 2)
        def _():
            pltpu.async_copy(_feat(h0 + 2), rows_a, sem_a)

        pltpu.make_async_copy(_feat(h0 + 1), rows_b, sem_b).wait()
        pltpu.sync_copy(rows_b, accum_sh.at[idx_v.at[h0 + 1]], add=True)
        return carry

    lax.fori_loop(0, _NHC // 2, _pair, 0)
    plsc.subcore_barrier()

    def _copy_out(dst_hbm):
        slices = [(sid * _CPT + j * _HC, _HC) for j in range(9)]
        slices.append((sid * _CPT + 576, 48))
        bufs = (rows_a, rows_b)
        sems = (sem_a, sem_b)

        def _hop2(k):
            off, size = slices[k]
            return (bufs[k % 2].at[pl.ds(0, size)],
                    dst_hbm.at[cid, pl.ds(off, size)], sems[k % 2])

        for k, (off, size) in enumerate(slices):
            if k >= 2:
                pltpu.make_async_copy(*_hop2(k - 2)).wait()
            pltpu.sync_copy(accum_sh.at[pl.ds(off, size)],
                            bufs[k % 2].at[pl.ds(0, size)])
            pltpu.async_copy(*_hop2(k))
        pltpu.make_async_copy(*_hop2(len(slices) - 2)).wait()
        pltpu.make_async_copy(*_hop2(len(slices) - 1)).wait()

        @pl.when(sid == 0)
        def _():
            pltpu.sync_copy(accum_sh.at[pl.ds(_NS * _CPT, _CREM)],
                            rows_a.at[pl.ds(0, _CREM)])
            pltpu.sync_copy(rows_a.at[pl.ds(0, _CREM)],
                            dst_hbm.at[cid, pl.ds(_NS * _CPT, _CREM)])

    _copy_out(psum_hbm)
    plsc.subcore_barrier()

    def _fill_ones(i, carry):
        for t in range(_D // 16):
            rows_a[i, pl.ds(t * 16, 16)] = ones16
        return carry

    lax.fori_loop(0, _HC, _fill_ones, 0)

    def _cfire(h, carry):
        pltpu.async_copy(rows_a, accum_sh.at[idx_v.at[h]], sem_a, add=True)
        return carry

    lax.fori_loop(0, _NHC, _cfire, 0)

    def _cdrain(h, carry):
        pltpu.make_async_copy(rows_a, accum_sh.at[idx_v.at[0]], sem_a).wait()
        return carry

    lax.fori_loop(0, _NHC, _cdrain, 0)
    plsc.subcore_barrier()
    _copy_out(pcnt_hbm)


@functools.cache
def _seg_call():
    return pl.kernel(
        _seg_body,
        out_type=(
            jax.ShapeDtypeStruct((_NC, _C, _D), jnp.float32),
            jax.ShapeDtypeStruct((_NC, _C, _D), jnp.float32),
        ),
        mesh=plsc.VectorSubcoreMesh(core_axis_name="c", subcore_axis_name="s",
                                    num_cores=_NC, num_subcores=_NS),
        scratch_types=[
            pltpu.VMEM((_NHC, _HC), jnp.int32),
            pltpu.VMEM((_HC, _D), jnp.float32),
            pltpu.VMEM((_HC, _D), jnp.float32),
            pltpu.VMEM_SHARED((_C, _D), jnp.float32),
            pltpu.SemaphoreType.DMA,
            pltpu.SemaphoreType.DMA,
        ],
    )


_BLK = 1000


def _loss_body(psum_ref, pcnt_ref, proto_ref, dphi_ref, loss_ref, acc_ref):
    i = pl.program_id(0)

    @pl.when(i == 0)
    def _():
        acc_ref[0] = 0.0
        acc_ref[1] = 0.0

    sums = psum_ref[0] + psum_ref[1]
    counts = (pcnt_ref[0] - psum_ref[0]
              + pcnt_ref[1] - psum_ref[1])[:, 0]
    present = counts > 0.0
    means = sums / jnp.maximum(counts, 1.0)[:, None]
    diff = means - (proto_ref[...] + dphi_ref[...])
    norms = jnp.sqrt(jnp.sum(diff * diff, axis=1))
    acc_ref[0] += jnp.sum(jnp.where(present, norms, 0.0))
    acc_ref[1] += jnp.sum(present.astype(jnp.float32))

    @pl.when(i == pl.num_programs(0) - 1)
    def _():
        val = acc_ref[0] / jnp.maximum(acc_ref[1], 1.0)
        loss_ref[...] = jnp.broadcast_to(val, (1, 1))


def _loss_call(psum, pcnt, prototypes, delta_phi):
    return pl.pallas_call(
        _loss_body,
        grid=(_C // _BLK,),
        in_specs=[
            pl.BlockSpec((_NC, _BLK, _D), lambda i: (0, i, 0)),
            pl.BlockSpec((_NC, _BLK, _D), lambda i: (0, i, 0)),
            pl.BlockSpec((_BLK, _D), lambda i: (i, 0)),
            pl.BlockSpec((_BLK, _D), lambda i: (i, 0)),
        ],
        out_specs=pl.BlockSpec((1, 1), lambda i: (0, 0)),
        out_shape=jax.ShapeDtypeStruct((1, 1), jnp.float32),
        scratch_shapes=[pltpu.SMEM((2,), jnp.float32)],
    )(psum, pcnt, prototypes, delta_phi)


@jax.jit
def kernel(t_features, t_pseudo_labels, prototypes, delta_phi):
    labels = t_pseudo_labels.reshape(_NW, _NHC, _HC)
    zeros = jnp.zeros((_HC, _D), jnp.float32)
    psum, pcnt = _seg_call()(t_features, labels, zeros)
    loss = _loss_call(psum, pcnt, prototypes, delta_phi)
    return loss[0, 0]
